# baseline (speedup 1.0000x reference)
import functools

import jax
import jax.numpy as jnp
from jax import lax
from jax.experimental import pallas as pl
from jax.experimental.pallas import tpu as pltpu
from jax.experimental.pallas import tpu_sc as plsc

NC = 2
NS = 16
NW = NC * NS
L = 16


@functools.lru_cache(maxsize=None)
def _make_kernel(B: int, D: int):
    rpw = B // NW
    C = 128
    sizes = [32, 96] + [C] * ((rpw - 256) // C) + [96, 32]
    offs = [sum(sizes[:i]) for i in range(len(sizes))]
    nch = len(sizes)
    nseg = D // L

    mesh = plsc.VectorSubcoreMesh(core_axis_name="c", subcore_axis_name="s")
    NBUF = 2

    @functools.partial(
        pl.kernel,
        mesh=mesh,
        out_type=jax.ShapeDtypeStruct((B,), jnp.float32),
        compiler_params=pltpu.CompilerParams(
            needs_layout_passes=False,
            disable_bounds_checks=True,
            disable_semaphore_checks=True,
            skip_device_barrier=True,
        ),
        scratch_types=[
            pltpu.VMEM((rpw,), jnp.int32),
            pltpu.VMEM((rpw,), jnp.int32),
            pltpu.VMEM((NBUF, C, D), jnp.float32),
            pltpu.VMEM((NBUF, C, D), jnp.float32),
            pltpu.VMEM((D,), jnp.float32),
            pltpu.VMEM((rpw,), jnp.float32),
            pltpu.VMEM((L, L + 1), jnp.float32),
            pltpu.SemaphoreType.DMA,
            pltpu.SemaphoreType.DMA,
            pltpu.SemaphoreType.DMA,
            pltpu.SemaphoreType.DMA,
            pltpu.SemaphoreType.DMA,
            pltpu.SemaphoreType.DMA,
        ],
    )
    def body(uidx_hbm, iidx_hbm, ut_hbm, it_hbm, w_hbm, out_hbm,
             uidx_v, iidx_v, urows, irows, w_v, out_v, tscr,
             sem_u0, sem_u1, sem_u2, sem_i0, sem_i1, sem_i2):
        wid = lax.axis_index("s") * NC + lax.axis_index("c")
        sem_u = (sem_u0, sem_u1, sem_u2)
        sem_i = (sem_i0, sem_i1, sem_i2)
        def start(c):
            buf = c % NBUF
            off, sz = offs[c], sizes[c]
            return (
                pltpu.async_copy(ut_hbm.at[uidx_v.at[pl.ds(off, sz)]],
                                 urows.at[buf].at[pl.ds(0, sz)], sem_u[buf]),
                pltpu.async_copy(it_hbm.at[iidx_v.at[pl.ds(off, sz)]],
                                 irows.at[buf].at[pl.ds(0, sz)], sem_i[buf]),
            )

        head = offs[NBUF] if NBUF < nch else rpw
        pltpu.sync_copy(uidx_hbm.at[pl.ds(wid * rpw, head)],
                        uidx_v.at[pl.ds(0, head)])
        pltpu.sync_copy(iidx_hbm.at[pl.ds(wid * rpw, head)],
                        iidx_v.at[pl.ds(0, head)])
        cps = {c: start(c) for c in range(min(NBUF, nch))}
        pltpu.sync_copy(uidx_hbm.at[pl.ds(wid * rpw + head, rpw - head)],
                        uidx_v.at[pl.ds(head, rpw - head)])
        pltpu.sync_copy(iidx_hbm.at[pl.ds(wid * rpw + head, rpw - head)],
                        iidx_v.at[pl.ds(head, rpw - head)])
        pltpu.sync_copy(w_hbm.at[0], w_v)
        lane = jnp.arange(L, dtype=jnp.int32)
        wsegs = [w_v[pl.ds(s * L, L)] for s in range(nseg)]
        for c in range(nch):
            buf = c % NBUF
            for cp in cps.pop(c):
                cp.wait()
            ub = urows.at[buf]
            ib = irows.at[buf]

            off = offs[c]

            def group(g, _, ub=ub, ib=ib, off=off):
                for k in range(L):
                    r = g * L + k
                    acc = (ub[r, pl.ds(0, L)] * ib[r, pl.ds(0, L)]) * wsegs[0]
                    for s in range(1, nseg):
                        acc = acc + (ub[r, pl.ds(s * L, L)]
                                     * ib[r, pl.ds(s * L, L)]) * wsegs[s]
                    tscr[k, pl.ds(0, L)] = acc
                cols = [
                    plsc.load_gather(
                        tscr, [lane, jnp.full((L,), j, dtype=jnp.int32)])
                    for j in range(L)
                ]
                while len(cols) > 1:
                    cols = [cols[i] + cols[i + 1]
                            for i in range(0, len(cols), 2)]
                out_v[pl.ds(off + g * L, L)] = cols[0]
                return 0

            lax.fori_loop(0, sizes[c] // L, group, 0)
            if c + NBUF < nch:
                cps[c + NBUF] = start(c + NBUF)

        pltpu.sync_copy(out_v, out_hbm.at[pl.ds(wid * rpw, rpw)])

    return body


def kernel(user_index, item_index, user_table, item_table, W):
    B = user_index.shape[0]
    D = user_table.shape[1]
    return _make_kernel(B, D)(
        user_index.astype(jnp.int32), item_index.astype(jnp.int32),
        user_table, item_table, W)


# speedup vs baseline: 1.0131x; 1.0131x over previous
import functools

import jax
import jax.numpy as jnp
from jax import lax
from jax.experimental import pallas as pl
from jax.experimental.pallas import tpu as pltpu
from jax.experimental.pallas import tpu_sc as plsc

NC = 2
NS = 16
NW = NC * NS
L = 16


@functools.lru_cache(maxsize=None)
def _make_kernel(B: int, D: int):
    rpw = B // NW
    C = 128
    sizes = [C] * (rpw // C)
    offs = [sum(sizes[:i]) for i in range(len(sizes))]
    nch = len(sizes)
    nseg = D // L

    mesh = plsc.VectorSubcoreMesh(core_axis_name="c", subcore_axis_name="s")
    NBUF = 2

    @functools.partial(
        pl.kernel,
        mesh=mesh,
        out_type=jax.ShapeDtypeStruct((B,), jnp.float32),
        compiler_params=pltpu.CompilerParams(
            needs_layout_passes=False,
            disable_bounds_checks=True,
            disable_semaphore_checks=True,
            skip_device_barrier=True,
        ),
        scratch_types=[
            pltpu.VMEM((rpw,), jnp.int32),
            pltpu.VMEM((rpw,), jnp.int32),
            pltpu.VMEM((NBUF, C, D), jnp.float32),
            pltpu.VMEM((NBUF, C, D), jnp.float32),
            pltpu.VMEM((D,), jnp.float32),
            pltpu.VMEM((rpw,), jnp.float32),
            pltpu.VMEM((L, L + 1), jnp.float32),
            pltpu.SemaphoreType.DMA,
            pltpu.SemaphoreType.DMA,
            pltpu.SemaphoreType.DMA,
            pltpu.SemaphoreType.DMA,
            pltpu.SemaphoreType.DMA,
            pltpu.SemaphoreType.DMA,
        ],
    )
    def body(uidx_hbm, iidx_hbm, ut_hbm, it_hbm, w_hbm, out_hbm,
             uidx_v, iidx_v, urows, irows, w_v, out_v, tscr,
             sem_u0, sem_u1, sem_u2, sem_i0, sem_i1, sem_i2):
        wid = lax.axis_index("s") * NC + lax.axis_index("c")
        sem_u = (sem_u0, sem_u1, sem_u2)
        sem_i = (sem_i0, sem_i1, sem_i2)
        def start(c):
            buf = c % NBUF
            off, sz = offs[c], sizes[c]
            return (
                pltpu.async_copy(ut_hbm.at[uidx_v.at[pl.ds(off, sz)]],
                                 urows.at[buf].at[pl.ds(0, sz)], sem_u[buf]),
                pltpu.async_copy(it_hbm.at[iidx_v.at[pl.ds(off, sz)]],
                                 irows.at[buf].at[pl.ds(0, sz)], sem_i[buf]),
            )

        head = offs[NBUF] if NBUF < nch else rpw
        pltpu.sync_copy(uidx_hbm.at[pl.ds(wid * rpw, head)],
                        uidx_v.at[pl.ds(0, head)])
        pltpu.sync_copy(iidx_hbm.at[pl.ds(wid * rpw, head)],
                        iidx_v.at[pl.ds(0, head)])
        cps = {c: start(c) for c in range(min(NBUF, nch))}
        pltpu.sync_copy(uidx_hbm.at[pl.ds(wid * rpw + head, rpw - head)],
                        uidx_v.at[pl.ds(head, rpw - head)])
        pltpu.sync_copy(iidx_hbm.at[pl.ds(wid * rpw + head, rpw - head)],
                        iidx_v.at[pl.ds(head, rpw - head)])
        pltpu.sync_copy(w_hbm.at[0], w_v)
        lane = jnp.arange(L, dtype=jnp.int32)
        wsegs = [w_v[pl.ds(s * L, L)] for s in range(nseg)]
        for c in range(nch):
            buf = c % NBUF
            for cp in cps.pop(c):
                cp.wait()
            ub = urows.at[buf]
            ib = irows.at[buf]

            off = offs[c]

            def group(g, _, ub=ub, ib=ib, off=off):
                for k in range(L):
                    r = g * L + k
                    acc = (ub[r, pl.ds(0, L)] * ib[r, pl.ds(0, L)]) * wsegs[0]
                    for s in range(1, nseg):
                        acc = acc + (ub[r, pl.ds(s * L, L)]
                                     * ib[r, pl.ds(s * L, L)]) * wsegs[s]
                    tscr[k, pl.ds(0, L)] = acc
                cols = [
                    plsc.load_gather(
                        tscr, [lane, jnp.full((L,), j, dtype=jnp.int32)])
                    for j in range(L)
                ]
                while len(cols) > 1:
                    cols = [cols[i] + cols[i + 1]
                            for i in range(0, len(cols), 2)]
                out_v[pl.ds(off + g * L, L)] = cols[0]
                return 0

            lax.fori_loop(0, sizes[c] // L, group, 0)
            if c + NBUF < nch:
                cps[c + NBUF] = start(c + NBUF)

        pltpu.sync_copy(out_v, out_hbm.at[pl.ds(wid * rpw, rpw)])

    return body


def kernel(user_index, item_index, user_table, item_table, W):
    B = user_index.shape[0]
    D = user_table.shape[1]
    return _make_kernel(B, D)(
        user_index.astype(jnp.int32), item_index.astype(jnp.int32),
        user_table, item_table, W)


# speedup vs baseline: 1.0380x; 1.0246x over previous
import functools

import jax
import jax.numpy as jnp
from jax import lax
from jax.experimental import pallas as pl
from jax.experimental.pallas import tpu as pltpu
from jax.experimental.pallas import tpu_sc as plsc

NC = 2
NS = 16
NW = NC * NS
L = 16


@functools.lru_cache(maxsize=None)
def _make_kernel(B: int, D: int):
    rpw = B // NW
    C = 128
    sizes = [C] * (rpw // C)
    offs = [sum(sizes[:i]) for i in range(len(sizes))]
    nch = len(sizes)
    nseg = D // L

    mesh = plsc.VectorSubcoreMesh(core_axis_name="c", subcore_axis_name="s")
    NBUF = 2

    @functools.partial(
        pl.kernel,
        mesh=mesh,
        out_type=jax.ShapeDtypeStruct((B,), jnp.float32),
        compiler_params=pltpu.CompilerParams(
            needs_layout_passes=False,
            disable_bounds_checks=True,
            disable_semaphore_checks=True,
            skip_device_barrier=True,
        ),
        scratch_types=[
            pltpu.VMEM((rpw,), jnp.int32),
            pltpu.VMEM((rpw,), jnp.int32),
            pltpu.VMEM((NBUF, C, D), jnp.float32),
            pltpu.VMEM((NBUF, C, D), jnp.float32),
            pltpu.VMEM((D,), jnp.float32),
            pltpu.VMEM((rpw,), jnp.float32),
            pltpu.VMEM((L, L + 1), jnp.float32),
            pltpu.SemaphoreType.DMA,
            pltpu.SemaphoreType.DMA,
            pltpu.SemaphoreType.DMA,
            pltpu.SemaphoreType.DMA,
            pltpu.SemaphoreType.DMA,
            pltpu.SemaphoreType.DMA,
            pltpu.SemaphoreType.DMA,
        ],
    )
    def body(uidx_hbm, iidx_hbm, ut_hbm, it_hbm, w_hbm, out_hbm,
             uidx_v, iidx_v, urows, irows, w_v, out_v, tscr,
             sem_u0, sem_u1, sem_u2, sem_i0, sem_i1, sem_i2, sem_w):
        wid = lax.axis_index("s") * NC + lax.axis_index("c")
        sem_u = (sem_u0, sem_u1, sem_u2)
        sem_i = (sem_i0, sem_i1, sem_i2)
        def start(c):
            buf = c % NBUF
            off, sz = offs[c], sizes[c]
            return (
                pltpu.async_copy(ut_hbm.at[uidx_v.at[pl.ds(off, sz)]],
                                 urows.at[buf].at[pl.ds(0, sz)], sem_u[buf]),
                pltpu.async_copy(it_hbm.at[iidx_v.at[pl.ds(off, sz)]],
                                 irows.at[buf].at[pl.ds(0, sz)], sem_i[buf]),
            )

        head = offs[NBUF] if NBUF < nch else rpw
        cp_w = pltpu.async_copy(w_hbm.at[0], w_v, sem_w)
        cp_hu = pltpu.async_copy(uidx_hbm.at[pl.ds(wid * rpw, head)],
                                 uidx_v.at[pl.ds(0, head)], sem_u2)
        cp_hi = pltpu.async_copy(iidx_hbm.at[pl.ds(wid * rpw, head)],
                                 iidx_v.at[pl.ds(0, head)], sem_i2)
        cp_hu.wait()
        cp_hi.wait()
        cps = {c: start(c) for c in range(min(NBUF, nch))}
        cp_ru = pltpu.async_copy(
            uidx_hbm.at[pl.ds(wid * rpw + head, rpw - head)],
            uidx_v.at[pl.ds(head, rpw - head)], sem_u2)
        cp_ri = pltpu.async_copy(
            iidx_hbm.at[pl.ds(wid * rpw + head, rpw - head)],
            iidx_v.at[pl.ds(head, rpw - head)], sem_i2)
        cp_w.wait()
        lane = jnp.arange(L, dtype=jnp.int32)
        wsegs = [w_v[pl.ds(s * L, L)] for s in range(nseg)]
        cp_ru.wait()
        cp_ri.wait()
        for c in range(nch):
            buf = c % NBUF
            for cp in cps.pop(c):
                cp.wait()
            ub = urows.at[buf]
            ib = irows.at[buf]

            off = offs[c]

            def group(g, _, ub=ub, ib=ib, off=off):
                for k in range(L):
                    r = g * L + k
                    acc = (ub[r, pl.ds(0, L)] * ib[r, pl.ds(0, L)]) * wsegs[0]
                    for s in range(1, nseg):
                        acc = acc + (ub[r, pl.ds(s * L, L)]
                                     * ib[r, pl.ds(s * L, L)]) * wsegs[s]
                    tscr[k, pl.ds(0, L)] = acc
                cols = [
                    plsc.load_gather(
                        tscr, [lane, jnp.full((L,), j, dtype=jnp.int32)])
                    for j in range(L)
                ]
                while len(cols) > 1:
                    cols = [cols[i] + cols[i + 1]
                            for i in range(0, len(cols), 2)]
                out_v[pl.ds(off + g * L, L)] = cols[0]
                return 0

            lax.fori_loop(0, sizes[c] // L, group, 0)
            if c + NBUF < nch:
                cps[c + NBUF] = start(c + NBUF)

        pltpu.sync_copy(out_v, out_hbm.at[pl.ds(wid * rpw, rpw)])

    return body


def kernel(user_index, item_index, user_table, item_table, W):
    B = user_index.shape[0]
    D = user_table.shape[1]
    return _make_kernel(B, D)(
        user_index.astype(jnp.int32), item_index.astype(jnp.int32),
        user_table, item_table, W)
